# baseline (speedup 1.0000x reference)
import functools

import jax
import jax.numpy as jnp
from jax import lax
from jax.experimental import pallas as pl
from jax.experimental.pallas import tpu as pltpu
from jax.experimental.pallas import tpu_sc as plsc

D = 64
B = 16384
L = 16
NC = 2
NS = 16
NW = NC * NS
PER_W = B // NW
CHUNK = 128
N_CHUNKS = PER_W // CHUNK


def _make_sc_kernel():
    mesh = plsc.VectorSubcoreMesh(core_axis_name="c", subcore_axis_name="s")

    @functools.partial(
        pl.kernel,
        out_type=jax.ShapeDtypeStruct((NW, L), jnp.float32),
        mesh=mesh,
        compiler_params=pltpu.CompilerParams(use_tc_tiling_on_sc=False),
        scratch_types=[
            pltpu.VMEM((CHUNK,), jnp.int32),
            pltpu.VMEM((CHUNK,), jnp.int32),
            pltpu.VMEM((CHUNK,), jnp.int32),
            pltpu.VMEM((CHUNK, D), jnp.float32),
            pltpu.VMEM((CHUNK, D), jnp.float32),
            pltpu.VMEM((CHUNK, D), jnp.float32),
            pltpu.VMEM((CHUNK, D), jnp.float32),
            pltpu.VMEM((CHUNK, D), jnp.float32),
            pltpu.VMEM((CHUNK, D), jnp.float32),
            pltpu.VMEM((L,), jnp.float32),
            pltpu.SemaphoreType.DMA,
        ],
    )
    def sc_kernel(heads, rels, tails, ere, eim, rre, rim, out,
                  hidx, ridx, tidx, hr, hi, tr, ti, rr, ri, accv, sem):
        wid = lax.axis_index("s") * NC + lax.axis_index("c")
        base = wid * PER_W

        accs = tuple(jnp.zeros((L,), jnp.float32) for _ in range(D // L))
        for ck in range(N_CHUNKS):
            off = base + ck * CHUNK
            pltpu.sync_copy(heads.at[pl.ds(off, CHUNK)], hidx)
            pltpu.sync_copy(rels.at[pl.ds(off, CHUNK)], ridx)
            pltpu.sync_copy(tails.at[pl.ds(off, CHUNK)], tidx)
            pltpu.async_copy(ere.at[hidx], hr, sem).wait()
            pltpu.async_copy(eim.at[hidx], hi, sem).wait()
            pltpu.async_copy(ere.at[tidx], tr, sem).wait()
            pltpu.async_copy(eim.at[tidx], ti, sem).wait()
            pltpu.async_copy(rre.at[ridx], rr, sem).wait()
            pltpu.async_copy(rim.at[ridx], ri, sem).wait()

            def row(i, a):
                new = []
                for j in range(D // L):
                    sl = pl.ds(j * L, L)
                    vhr = hr[i, sl]
                    vhi = hi[i, sl]
                    vtr = tr[i, sl]
                    vti = ti[i, sl]
                    vrr = rr[i, sl]
                    vri = ri[i, sl]
                    v = vrr * (vhr * vtr + vhi * vti) + vri * (vhr * vti - vhi * vtr)
                    new.append(a[j] + v)
                return tuple(new)

            accs = lax.fori_loop(0, CHUNK, row, accs)

        total = accs[0]
        for j in range(1, D // L):
            total = total + accs[j]
        accv[...] = total
        pltpu.sync_copy(accv, out.at[wid])

    return sc_kernel


_sc_score = _make_sc_kernel()


def kernel(heads, rels, tails, entity_re, entity_im, r_re, r_im):
    parts = _sc_score(
        heads.astype(jnp.int32),
        rels.astype(jnp.int32),
        tails.astype(jnp.int32),
        entity_re, entity_im, r_re, r_im,
    )
    return jnp.sum(parts)


# speedup vs baseline: 1.2143x; 1.2143x over previous
import functools

import jax
import jax.numpy as jnp
from jax import lax
from jax.experimental import pallas as pl
from jax.experimental.pallas import tpu as pltpu
from jax.experimental.pallas import tpu_sc as plsc

D = 64
D2 = 128
B = 16384
L = 16
NC = 2
NS = 16
NW = NC * NS
PER_W = B // NW
CHUNK = 128
N_CHUNKS = PER_W // CHUNK


def _make_sc_kernel():
    mesh = plsc.VectorSubcoreMesh(core_axis_name="c", subcore_axis_name="s")

    @functools.partial(
        pl.kernel,
        out_type=jax.ShapeDtypeStruct((NW, L), jnp.float32),
        mesh=mesh,
        scratch_types=[
            pltpu.VMEM((CHUNK,), jnp.int32),
            pltpu.VMEM((CHUNK,), jnp.int32),
            pltpu.VMEM((CHUNK,), jnp.int32),
            pltpu.VMEM((CHUNK, D2), jnp.float32),
            pltpu.VMEM((CHUNK, D2), jnp.float32),
            pltpu.VMEM((CHUNK, D2), jnp.float32),
            pltpu.VMEM((L,), jnp.float32),
            pltpu.SemaphoreType.DMA,
        ],
    )
    def sc_kernel(heads, rels, tails, ent, rel, out,
                  hidx, ridx, tidx, hbuf, tbuf, rbuf, accv, sem):
        wid = lax.axis_index("s") * NC + lax.axis_index("c")
        base = wid * PER_W

        def chunk_body(ck, accs):
            off = base + ck * CHUNK
            pltpu.sync_copy(heads.at[pl.ds(off, CHUNK)], hidx)
            pltpu.sync_copy(rels.at[pl.ds(off, CHUNK)], ridx)
            pltpu.sync_copy(tails.at[pl.ds(off, CHUNK)], tidx)
            pltpu.async_copy(ent.at[hidx], hbuf, sem).wait()
            pltpu.async_copy(ent.at[tidx], tbuf, sem).wait()
            pltpu.async_copy(rel.at[ridx], rbuf, sem).wait()

            def row(i, a):
                new = []
                for j in range(D // L):
                    re_sl = pl.ds(j * L, L)
                    im_sl = pl.ds(D + j * L, L)
                    vhr = hbuf[i, re_sl]
                    vhi = hbuf[i, im_sl]
                    vtr = tbuf[i, re_sl]
                    vti = tbuf[i, im_sl]
                    vrr = rbuf[i, re_sl]
                    vri = rbuf[i, im_sl]
                    v = (vrr * (vhr * vtr + vhi * vti)
                         + vri * (vhr * vti - vhi * vtr))
                    new.append(a[j] + v)
                return tuple(new)

            return lax.fori_loop(0, CHUNK, row, accs)

        accs = lax.fori_loop(
            0, N_CHUNKS, chunk_body,
            tuple(jnp.zeros((L,), jnp.float32) for _ in range(D // L)))
        total = accs[0]
        for j in range(1, D // L):
            total = total + accs[j]
        accv[...] = total
        pltpu.sync_copy(accv, out.at[wid])

    return sc_kernel


_sc_score = _make_sc_kernel()


def kernel(heads, rels, tails, entity_re, entity_im, r_re, r_im):
    ent = jnp.concatenate([entity_re, entity_im], axis=1)
    rel = jnp.concatenate([r_re, r_im], axis=1)
    parts = _sc_score(
        heads.astype(jnp.int32),
        rels.astype(jnp.int32),
        tails.astype(jnp.int32),
        ent, rel,
    )
    return jnp.sum(parts)
